# baseline (speedup 1.0000x reference)
import functools

import numpy as np

import jax
import jax.numpy as jnp
from jax import lax
from jax.experimental import pallas as pl
from jax.experimental.pallas import tpu as pltpu
from jax.experimental.pallas import tpu_sc as plsc

_N = 10000
_E = 320000
_D = 128

_NC = 2
_NS = 16
_NW = _NC * _NS
_EPW = _E // _NW
_CH = 80
_NCHUNK = _EPW // _CH
_G = _CH // 16
_NB = 4
_RPW = 624
_TAIL = _N - _NS * _RPW

_mesh = plsc.VectorSubcoreMesh(core_axis_name="c", subcore_axis_name="s")

_idx_scratch = [pltpu.VMEM((_CH,), jnp.int32) for _ in range(2 * _NB)]
_ev_scratch = [pltpu.VMEM((_CH,), jnp.float32) for _ in range(_NB)]
_row_scratch = [pltpu.VMEM((_CH, _D), jnp.float32) for _ in range(_NB)]
_sem_scratch = [pltpu.SemaphoreType.DMA for _ in range(5 * _NB)]


@functools.partial(
    pl.kernel,
    out_type=jax.ShapeDtypeStruct((_NC, _N, _D), jnp.float32),
    mesh=_mesh,
    scratch_types=(
        [pltpu.VMEM_SHARED((_N, _D), jnp.float32)]
        + _idx_scratch + _ev_scratch + _row_scratch + _sem_scratch
    ),
)
def _sc_aggregate(src_hbm, dst_hbm, ev_hbm, ego_hbm, out_hbm, acc, *scratch):
    srcbs = scratch[0:_NB]
    dstbs = scratch[_NB:2 * _NB]
    evbs = scratch[2 * _NB:3 * _NB]
    rowss = scratch[3 * _NB:4 * _NB]
    semg = scratch[4 * _NB:5 * _NB]
    sems = scratch[5 * _NB:6 * _NB]
    semsrc = scratch[6 * _NB:7 * _NB]
    semev = scratch[7 * _NB:8 * _NB]
    semd = scratch[8 * _NB:9 * _NB]

    c = lax.axis_index("c")
    s = lax.axis_index("s")
    wid = s * _NC + c
    rbase = s * _RPW
    ebase = wid * _EPW

    def esl(i):
        return pl.ds(ebase + i * _CH, _CH)

    def start_src(i, r):
        pltpu.async_copy(src_hbm.at[esl(i)], srcbs[r], semsrc[r])

    def wait_src(r):
        pltpu.make_async_copy(src_hbm.at[esl(0)], srcbs[r], semsrc[r]).wait()

    def start_ev(i, r):
        pltpu.async_copy(ev_hbm.at[esl(i)], evbs[r], semev[r])

    def wait_ev(r):
        pltpu.make_async_copy(ev_hbm.at[esl(0)], evbs[r], semev[r]).wait()

    def start_dst(i, r):
        pltpu.async_copy(dst_hbm.at[esl(i)], dstbs[r], semd[r])

    def wait_dst(r):
        pltpu.make_async_copy(dst_hbm.at[esl(0)], dstbs[r], semd[r]).wait()

    def start_gather(r):
        pltpu.async_copy(ego_hbm.at[srcbs[r]], rowss[r], semg[r])

    def wait_gather(r):
        pltpu.make_async_copy(ego_hbm.at[srcbs[r]], rowss[r], semg[r]).wait()

    def start_scatter(r):
        pltpu.async_copy(rowss[r], acc.at[dstbs[r]], sems[r], add=True)

    def wait_scatter(r):
        pltpu.make_async_copy(rowss[r], acc.at[dstbs[r]], sems[r]).wait()

    def load_idx_sync(i, r):
        pltpu.sync_copy(src_hbm.at[esl(i)], srcbs[r])
        pltpu.sync_copy(dst_hbm.at[esl(i)], dstbs[r])
        pltpu.sync_copy(ev_hbm.at[esl(i)], evbs[r])

    def scale_chunk(r):
        rr = rowss[r]
        evr = evbs[r]

        @plsc.parallel_loop(0, _G)
        def _(g):
            wvec = evr[pl.ds(g * 16, 16)]
            base = g * 16
            for l in range(16):
                w = lax.gather(
                    wvec, jnp.full((16, 1), l, jnp.int32),
                    lax.GatherDimensionNumbers(
                        offset_dims=(), collapsed_slice_dims=(0,),
                        start_index_map=(0,)),
                    (1,), mode=lax.GatherScatterMode.PROMISE_IN_BOUNDS)
                e = base + l
                for j in range(_D // 16):
                    sl = pl.ds(j * 16, 16)
                    rr[e, sl] = rr[e, sl] * w

    for i in range(_NB):
        load_idx_sync(i, i)
    start_gather(0)
    start_gather(1)

    @pl.when(c == 0)
    def _():
        pltpu.sync_copy(ego_hbm.at[pl.ds(rbase, _RPW)],
                        acc.at[pl.ds(rbase, _RPW)])

        @pl.when(s == 0)
        def _():
            pltpu.sync_copy(ego_hbm.at[pl.ds(_NS * _RPW, _TAIL)],
                            acc.at[pl.ds(_NS * _RPW, _TAIL)])

    @pl.when(c != 0)
    def _():
        zbuf = rowss[3]
        zvec = jnp.zeros((16,), jnp.float32)

        def zbody(e, carry):
            for j in range(_D // 16):
                zbuf[e, pl.ds(j * 16, 16)] = zvec
            return carry

        lax.fori_loop(0, _CH, zbody, 0)
        for k in range(7):
            pltpu.async_copy(zbuf, acc.at[pl.ds(rbase + k * _CH, _CH)],
                             semg[3])
        pltpu.async_copy(zbuf.at[pl.ds(0, 64)],
                         acc.at[pl.ds(rbase + 7 * _CH, 64)], semg[3])
        for k in range(7):
            pltpu.make_async_copy(
                zbuf, acc.at[pl.ds(rbase + k * _CH, _CH)], semg[3]).wait()
        pltpu.make_async_copy(
            zbuf.at[pl.ds(0, 64)],
            acc.at[pl.ds(rbase + 7 * _CH, 64)], semg[3]).wait()

        @pl.when(s == 0)
        def _():
            pltpu.sync_copy(zbuf.at[pl.ds(0, _TAIL)],
                            acc.at[pl.ds(_NS * _RPW, _TAIL)])

    plsc.subcore_barrier()

    def one_chunk(i, r, *, drain=True, idx2=True, gather2=True,
                  wait_idx=True, start4=True):
        r2 = (r + 2) % _NB
        if drain:
            wait_scatter(r2)
        if idx2:
            start_dst(i + 2, r2)
            wait_src(r2)
        if gather2:
            start_gather(r2)
        wait_gather(r)
        if wait_idx:
            wait_ev(r)
        scale_chunk(r)
        if wait_idx:
            wait_dst(r)
        start_scatter(r)
        if start4:
            start_src(i + 4, r)
            start_ev(i + 4, r)

    one_chunk(0, 0, drain=False, idx2=False, wait_idx=False)
    one_chunk(1, 1, drain=False, idx2=False, wait_idx=False)
    one_chunk(2, 2, wait_idx=False)
    one_chunk(3, 3, wait_idx=False)

    def ring_body(k, carry):
        i = 4 * k
        one_chunk(i, 0)
        one_chunk(i + 1, 1)
        one_chunk(i + 2, 2)
        one_chunk(i + 3, 3)
        return carry

    lax.fori_loop(1, _NCHUNK // _NB - 1, ring_body, 0)

    one_chunk(120, 0)
    one_chunk(121, 1, start4=False)
    one_chunk(122, 2, start4=False)
    one_chunk(123, 3, idx2=False, gather2=False, start4=False)
    one_chunk(124, 0, idx2=False, gather2=False, start4=False)

    wait_scatter(3)
    wait_scatter(0)

    plsc.subcore_barrier()

    pltpu.sync_copy(acc.at[pl.ds(rbase, _RPW)],
                    out_hbm.at[c, pl.ds(rbase, _RPW)])

    @pl.when(s == 0)
    def _():
        pltpu.sync_copy(acc.at[pl.ds(_NS * _RPW, _TAIL)],
                        out_hbm.at[c, pl.ds(_NS * _RPW, _TAIL)])


def _split_body(ei_ref, s_ref, d_ref):
    s_ref[...] = ei_ref[0]
    d_ref[...] = ei_ref[1]


def _tc_split(edge_index):
    return pl.pallas_call(
        _split_body,
        out_shape=[jax.ShapeDtypeStruct((_E,), jnp.int32),
                   jax.ShapeDtypeStruct((_E,), jnp.int32)],
    )(edge_index)


def _tc_body(p_ref, w_ref, b_ref, o_ref):
    x = p_ref[0] + p_ref[1]
    y = lax.dot_general(x, w_ref[...], (((1,), (1,)), ((), ())),
                        preferred_element_type=jnp.float32)
    y = y + b_ref[...]
    o_ref[...] = jnp.where(y >= 0, y, y * jnp.float32(0.01))


_BR = 2000


def _tc_dense(partial, w, b2):
    return pl.pallas_call(
        _tc_body,
        grid=(_N // _BR,),
        in_specs=[
            pl.BlockSpec((_NC, _BR, _D), lambda i: (0, i, 0)),
            pl.BlockSpec((_D, _D), lambda i: (0, 0)),
            pl.BlockSpec((1, _D), lambda i: (0, 0)),
        ],
        out_specs=pl.BlockSpec((_BR, _D), lambda i: (i, 0)),
        out_shape=jax.ShapeDtypeStruct((_N, _D), jnp.float32),
    )(partial, w, b2)


@jax.jit
def kernel(edge_index, edge_values, ego_embeddings, W, b):
    src, dst = _tc_split(edge_index)
    partial = _sc_aggregate(src, dst, edge_values, ego_embeddings)
    return _tc_dense(partial, W, b.reshape(1, _D))


# speedup vs baseline: 1.1541x; 1.1541x over previous
import functools

import numpy as np

import jax
import jax.numpy as jnp
from jax import lax
from jax.experimental import pallas as pl
from jax.experimental.pallas import tpu as pltpu
from jax.experimental.pallas import tpu_sc as plsc

_N = 10000
_E = 320000
_D = 128

_NC = 2
_NS = 16
_NW = _NC * _NS
_EPW = _E // _NW
_CH = 80
_NCHUNK = _EPW // _CH
_G = _CH // 16
_NB = 4
_RPW = 624
_TAIL = _N - _NS * _RPW

_mesh = plsc.VectorSubcoreMesh(core_axis_name="c", subcore_axis_name="s")

_idx_scratch = [pltpu.VMEM((_CH,), jnp.int32) for _ in range(2 * _NB)]
_ev_scratch = [pltpu.VMEM((_CH,), jnp.float32) for _ in range(_NB)]
_row_scratch = [pltpu.VMEM((_CH, _D), jnp.float32) for _ in range(_NB)]
_sem_scratch = [pltpu.SemaphoreType.DMA for _ in range(5 * _NB)]


@functools.partial(
    pl.kernel,
    out_type=jax.ShapeDtypeStruct((_NC, _N, _D), jnp.float32),
    mesh=_mesh,
    scratch_types=(
        [pltpu.VMEM_SHARED((_N, _D), jnp.float32)]
        + _idx_scratch + _ev_scratch + _row_scratch + _sem_scratch
    ),
)
def _sc_aggregate(src_hbm, dst_hbm, ev_hbm, ego_hbm, out_hbm, acc, *scratch):
    srcbs = scratch[0:_NB]
    dstbs = scratch[_NB:2 * _NB]
    evbs = scratch[2 * _NB:3 * _NB]
    rowss = scratch[3 * _NB:4 * _NB]
    semg = scratch[4 * _NB:5 * _NB]
    sems = scratch[5 * _NB:6 * _NB]
    semsrc = scratch[6 * _NB:7 * _NB]
    semev = scratch[7 * _NB:8 * _NB]
    semd = scratch[8 * _NB:9 * _NB]

    c = lax.axis_index("c")
    s = lax.axis_index("s")
    wid = s * _NC + c
    rbase = s * _RPW
    ebase = wid * _EPW

    def esl(i):
        return pl.ds(ebase + i * _CH, _CH)

    def start_src(i, r):
        pltpu.async_copy(src_hbm.at[esl(i)], srcbs[r], semsrc[r])

    def wait_src(r):
        pltpu.make_async_copy(src_hbm.at[esl(0)], srcbs[r], semsrc[r]).wait()

    def start_ev(i, r):
        pltpu.async_copy(ev_hbm.at[esl(i)], evbs[r], semev[r])

    def wait_ev(r):
        pltpu.make_async_copy(ev_hbm.at[esl(0)], evbs[r], semev[r]).wait()

    def start_dst(i, r):
        pltpu.async_copy(dst_hbm.at[esl(i)], dstbs[r], semd[r])

    def wait_dst(r):
        pltpu.make_async_copy(dst_hbm.at[esl(0)], dstbs[r], semd[r]).wait()

    def start_gather(r):
        pltpu.async_copy(ego_hbm.at[srcbs[r]], rowss[r], semg[r])

    def wait_gather(r):
        pltpu.make_async_copy(ego_hbm.at[srcbs[r]], rowss[r], semg[r]).wait()

    def start_scatter(r):
        pltpu.async_copy(rowss[r], acc.at[dstbs[r]], sems[r], add=True)

    def wait_scatter(r):
        pltpu.make_async_copy(rowss[r], acc.at[dstbs[r]], sems[r]).wait()

    def load_idx_sync(i, r):
        pltpu.sync_copy(src_hbm.at[esl(i)], srcbs[r])
        pltpu.sync_copy(dst_hbm.at[esl(i)], dstbs[r])
        pltpu.sync_copy(ev_hbm.at[esl(i)], evbs[r])

    def scale_chunk(r):
        rr = rowss[r]
        evr = evbs[r]

        def body(g, carry):
            wvec = evr[pl.ds(g * 16, 16)]
            base = g * 16
            for l in range(16):
                w = lax.gather(
                    wvec, jnp.full((16, 1), l, jnp.int32),
                    lax.GatherDimensionNumbers(
                        offset_dims=(), collapsed_slice_dims=(0,),
                        start_index_map=(0,)),
                    (1,), mode=lax.GatherScatterMode.PROMISE_IN_BOUNDS)
                e = base + l
                for j in range(_D // 16):
                    sl = pl.ds(j * 16, 16)
                    rr[e, sl] = rr[e, sl] * w
            return carry

        lax.fori_loop(0, _G, body, 0)

    for i in range(_NB):
        load_idx_sync(i, i)
    start_gather(0)
    start_gather(1)

    @pl.when(c == 0)
    def _():
        pltpu.sync_copy(ego_hbm.at[pl.ds(rbase, _RPW)],
                        acc.at[pl.ds(rbase, _RPW)])

        @pl.when(s == 0)
        def _():
            pltpu.sync_copy(ego_hbm.at[pl.ds(_NS * _RPW, _TAIL)],
                            acc.at[pl.ds(_NS * _RPW, _TAIL)])

    @pl.when(c != 0)
    def _():
        zbuf = rowss[3]
        zvec = jnp.zeros((16,), jnp.float32)

        def zbody(e, carry):
            for j in range(_D // 16):
                zbuf[e, pl.ds(j * 16, 16)] = zvec
            return carry

        lax.fori_loop(0, _CH, zbody, 0)
        for k in range(7):
            pltpu.async_copy(zbuf, acc.at[pl.ds(rbase + k * _CH, _CH)],
                             semg[3])
        pltpu.async_copy(zbuf.at[pl.ds(0, 64)],
                         acc.at[pl.ds(rbase + 7 * _CH, 64)], semg[3])
        for k in range(7):
            pltpu.make_async_copy(
                zbuf, acc.at[pl.ds(rbase + k * _CH, _CH)], semg[3]).wait()
        pltpu.make_async_copy(
            zbuf.at[pl.ds(0, 64)],
            acc.at[pl.ds(rbase + 7 * _CH, 64)], semg[3]).wait()

        @pl.when(s == 0)
        def _():
            pltpu.sync_copy(zbuf.at[pl.ds(0, _TAIL)],
                            acc.at[pl.ds(_NS * _RPW, _TAIL)])

    plsc.subcore_barrier()

    def one_chunk(i, r, *, drain=True, idx2=True, gather2=True,
                  wait_idx=True, start4=True):
        r2 = (r + 2) % _NB
        if drain:
            wait_scatter(r2)
        if idx2:
            start_dst(i + 2, r2)
            wait_src(r2)
        if gather2:
            start_gather(r2)
        wait_gather(r)
        if wait_idx:
            wait_ev(r)
        scale_chunk(r)
        if wait_idx:
            wait_dst(r)
        start_scatter(r)
        if start4:
            start_src(i + 4, r)
            start_ev(i + 4, r)

    one_chunk(0, 0, drain=False, idx2=False, wait_idx=False)
    one_chunk(1, 1, drain=False, idx2=False, wait_idx=False)
    one_chunk(2, 2, wait_idx=False)
    one_chunk(3, 3, wait_idx=False)

    def ring_body(k, carry):
        i = 4 * k
        one_chunk(i, 0)
        one_chunk(i + 1, 1)
        one_chunk(i + 2, 2)
        one_chunk(i + 3, 3)
        return carry

    lax.fori_loop(1, _NCHUNK // _NB - 1, ring_body, 0)

    one_chunk(120, 0)
    one_chunk(121, 1, start4=False)
    one_chunk(122, 2, start4=False)
    one_chunk(123, 3, idx2=False, gather2=False, start4=False)
    one_chunk(124, 0, idx2=False, gather2=False, start4=False)

    wait_scatter(3)
    wait_scatter(0)

    plsc.subcore_barrier()

    pltpu.sync_copy(acc.at[pl.ds(rbase, _RPW)],
                    out_hbm.at[c, pl.ds(rbase, _RPW)])

    @pl.when(s == 0)
    def _():
        pltpu.sync_copy(acc.at[pl.ds(_NS * _RPW, _TAIL)],
                        out_hbm.at[c, pl.ds(_NS * _RPW, _TAIL)])


def _split_body(ei_ref, s_ref, d_ref):
    s_ref[...] = ei_ref[0]
    d_ref[...] = ei_ref[1]


def _tc_split(edge_index):
    return pl.pallas_call(
        _split_body,
        out_shape=[jax.ShapeDtypeStruct((_E,), jnp.int32),
                   jax.ShapeDtypeStruct((_E,), jnp.int32)],
    )(edge_index)


def _tc_body(p_ref, w_ref, b_ref, o_ref):
    x = p_ref[0] + p_ref[1]
    y = lax.dot_general(x, w_ref[...], (((1,), (1,)), ((), ())),
                        preferred_element_type=jnp.float32)
    y = y + b_ref[...]
    o_ref[...] = jnp.where(y >= 0, y, y * jnp.float32(0.01))


_BR = 2000


def _tc_dense(partial, w, b2):
    return pl.pallas_call(
        _tc_body,
        grid=(_N // _BR,),
        in_specs=[
            pl.BlockSpec((_NC, _BR, _D), lambda i: (0, i, 0)),
            pl.BlockSpec((_D, _D), lambda i: (0, 0)),
            pl.BlockSpec((1, _D), lambda i: (0, 0)),
        ],
        out_specs=pl.BlockSpec((_BR, _D), lambda i: (i, 0)),
        out_shape=jax.ShapeDtypeStruct((_N, _D), jnp.float32),
    )(partial, w, b2)


@jax.jit
def kernel(edge_index, edge_values, ego_embeddings, W, b):
    src, dst = _tc_split(edge_index)
    partial = _sc_aggregate(src, dst, edge_values, ego_embeddings)
    return _tc_dense(partial, W, b.reshape(1, _D))
